# baseline (speedup 1.0000x reference)
import functools

import jax
import jax.numpy as jnp
from jax import lax
from jax.experimental import pallas as pl
from jax.experimental.pallas import tpu as pltpu
from jax.experimental.pallas import tpu_sc as plsc


def _lr_expand_sc(x2d, idx2d):
    NCH, CH = idx2d.shape
    Dd = x2d.shape[1]
    R = NCH * CH
    info = plsc.get_sparse_core_info()
    NC, NS = info.num_cores, info.num_subcores
    NW = NC * NS
    CPW = NCH // NW
    mesh = plsc.VectorSubcoreMesh(core_axis_name="c", subcore_axis_name="s")

    S = x2d.shape[0]
    SPT = S // NS
    G = 16
    scratch = [pltpu.VMEM_SHARED((S, Dd), jnp.float32),
               pltpu.VMEM((CPW, CH), jnp.int32)]
    scratch += [pltpu.VMEM((CH, Dd), jnp.float32) for _ in range(CPW)]
    scratch += [pltpu.SemaphoreType.DMA for _ in range(2 * CPW)]

    @functools.partial(
        pl.kernel, mesh=mesh,
        out_type=jax.ShapeDtypeStruct((R, Dd), jnp.float32),
        scratch_types=scratch,
        compiler_params=pltpu.CompilerParams(needs_layout_passes=False),
    )
    def k(x_hbm, idx_hbm, out_hbm, x_sh, idx_v, *bufs_and_sems):
        rows = bufs_and_sems[:CPW]
        sg = bufs_and_sems[CPW:2 * CPW]
        so = bufs_and_sems[2 * CPW:]
        sid = lax.axis_index("s")
        wid = sid * NC + lax.axis_index("c")
        c0 = wid * CPW
        pltpu.sync_copy(x_hbm.at[pl.ds(sid * SPT, SPT)],
                        x_sh.at[pl.ds(sid * SPT, SPT)])
        pltpu.sync_copy(idx_hbm.at[pl.ds(c0, CPW)], idx_v)
        plsc.subcore_barrier()
        outs = []
        for ci in range(CPW):
            pend = []
            for g in range(CH // G):
                v = idx_v[ci, pl.ds(g * G, G)]
                grp = [pltpu.async_copy(
                           x_sh.at[pl.ds(v[j], 1)],
                           rows[ci].at[pl.ds(g * G + j, 1)], sg[ci])
                       for j in range(G)]
                for p in pend:
                    p.wait()
                pend = grp
            for p in pend:
                p.wait()
            outs.append(pltpu.async_copy(
                rows[ci], out_hbm.at[pl.ds((c0 + ci) * CH, CH)], so[ci]))
        for o in outs:
            o.wait()

    return k(x2d, idx2d)


def _route_body(dur_ref, out_ref):
    t = dur_ref.shape[1]
    m = out_ref.shape[2]
    b = pl.program_id(0)
    dcol = dur_ref[0].astype(jnp.float32)
    ii = lax.broadcasted_iota(jnp.int32, (t, t), 0)
    jj = lax.broadcasted_iota(jnp.int32, (t, t), 1)
    tril = (jj <= ii).astype(jnp.float32)
    cumt = jnp.dot(tril, dcol, preferred_element_type=jnp.float32)
    frames = lax.broadcasted_iota(jnp.int32, (1, m), 1).astype(jnp.float32)
    cmpt = (cumt <= frames).astype(jnp.float32)
    ones = jnp.ones((1, t), jnp.float32)
    tok = jnp.dot(ones, cmpt, preferred_element_type=jnp.float32).astype(jnp.int32)
    out_ref[0] = jnp.minimum(tok, t - 1) + b * t


def _route_call(dur, M):
    B, T = dur.shape
    out = pl.pallas_call(
        _route_body,
        grid=(B,),
        in_specs=[pl.BlockSpec((1, T, 1), lambda i: (i, 0, 0))],
        out_specs=pl.BlockSpec((1, 1, M), lambda i: (i, 0, 0)),
        out_shape=jax.ShapeDtypeStruct((B, 1, M), jnp.int32),
    )(dur.reshape(B, T, 1))
    return out.reshape(B, M)


def _ln(h, s, b):
    mu = jnp.mean(h, axis=1, keepdims=True)
    d = h - mu
    var = jnp.mean(d * d, axis=1, keepdims=True)
    return d * lax.rsqrt(var + 1e-5) * s + b


def _conv3(h, w_ref, b_ref):
    a = jnp.dot(h, w_ref[0], preferred_element_type=jnp.float32)
    c = jnp.dot(h, w_ref[1], preferred_element_type=jnp.float32)
    e = jnp.dot(h, w_ref[2], preferred_element_type=jnp.float32)
    n = h.shape[0]
    z = jnp.zeros((1, a.shape[1]), jnp.float32)
    return (jnp.concatenate([z, a[: n - 1]], axis=0) + c
            + jnp.concatenate([e[1:], z], axis=0) + b_ref[...])


def _mlp(h, w1, b1, s1, g1, w2, b2, s2, g2, wl, bl):
    h = _ln(jnp.maximum(_conv3(h, w1, b1), 0.0), s1[...], g1[...])
    h = _ln(jnp.maximum(_conv3(h, w2, b2), 0.0), s2[...], g2[...])
    return jnp.dot(h, wl[...], preferred_element_type=jnp.float32) + bl[...]


def _wargs(p):
    d = p['b1'].shape[0]
    return (p['w1'], p['b1'].reshape(1, d), p['ln1_s'].reshape(1, d),
            p['ln1_b'].reshape(1, d), p['w2'], p['b2'].reshape(1, d),
            p['ln2_s'].reshape(1, d), p['ln2_b'].reshape(1, d),
            p['wl'], p['bl'].reshape(1, 1))


def _wspecs(d):
    def wspec(shape):
        return pl.BlockSpec(shape, lambda i: tuple(0 for _ in shape))
    return [wspec((3, d, d)), wspec((1, d)), wspec((1, d)), wspec((1, d)),
            wspec((3, d, d)), wspec((1, d)), wspec((1, d)), wspec((1, d)),
            wspec((d, 1)), wspec((1, 1))]


def _dur_body(x_ref, w1, b1, s1, g1, w2, b2, s2, g2, wl, bl, out_ref):
    t = x_ref.shape[0]
    p = _mlp(x_ref[...], w1, b1, s1, g1, w2, b2, s2, g2, wl, bl)
    out_ref[0] = p.reshape(1, t)


def _dur_call(x2d, B, p):
    R, Dd = x2d.shape
    T = R // B
    out = pl.pallas_call(
        _dur_body,
        grid=(B,),
        in_specs=[pl.BlockSpec((T, Dd), lambda i: (i, 0))] + _wspecs(Dd),
        out_specs=pl.BlockSpec((1, 1, T), lambda i: (i, 0, 0)),
        out_shape=jax.ShapeDtypeStruct((B, 1, T), jnp.float32),
    )(x2d, *_wargs(p))
    return out.reshape(B, T)


def _emb_add(tgt_ref, q1_ref, q2_ref, tab_ref):
    v = tgt_ref[...]
    oh = ((q1_ref[...] < v) & (v <= q2_ref[...])).astype(jnp.float32)
    return jnp.dot(oh, tab_ref[...], preferred_element_type=jnp.float32)


def _pitch_body(xe_ref, dur_ref, ml_ref, tgt_ref, q1_ref, q2_ref, tab_ref,
                w1, b1, s1, g1, w2, b2, s2, g2, wl, bl, pred_ref, xout_ref):
    m = xe_ref.shape[0]
    bound = jnp.minimum(jnp.sum(dur_ref[...]), ml_ref[0, 0])
    io = lax.broadcasted_iota(jnp.int32, (m, 1), 0)
    h = xe_ref[...] * (io < bound).astype(jnp.float32)
    pred_ref[0] = _mlp(h, w1, b1, s1, g1, w2, b2, s2, g2, wl, bl).reshape(1, m)
    xout_ref[...] = h + _emb_add(tgt_ref, q1_ref, q2_ref, tab_ref)


def _energy_body(xe_ref, tgt_ref, q1_ref, q2_ref, tab_ref,
                 w1, b1, s1, g1, w2, b2, s2, g2, wl, bl, pred_ref, xout_ref):
    m = xe_ref.shape[0]
    h = xe_ref[...]
    pred_ref[0] = _mlp(h, w1, b1, s1, g1, w2, b2, s2, g2, wl, bl).reshape(1, m)
    xout_ref[...] = h + _emb_add(tgt_ref, q1_ref, q2_ref, tab_ref)


def _quant_bounds(quant):
    q1 = jnp.concatenate([jnp.full((1,), -jnp.inf, jnp.float32), quant])
    q2 = jnp.concatenate([quant, jnp.full((1,), jnp.inf, jnp.float32)])
    return q1.reshape(1, -1), q2.reshape(1, -1)


def _var_call(xe2d, B, dur, max_len, tgt, quant, tab, p):
    R, Dd = xe2d.shape
    M = R // B
    nb = tab.shape[0]
    q1, q2 = _quant_bounds(quant)
    xspec = pl.BlockSpec((M, Dd), lambda i: (i, 0))
    qspec = pl.BlockSpec((1, nb), lambda i: (0, 0))
    in_specs = [xspec]
    args = [xe2d]
    body = _energy_body
    if dur is not None:
        T = dur.shape[1]
        ml = jnp.asarray(max_len, jnp.int32).reshape(1, 1)
        in_specs += [pl.BlockSpec((1, 1, T), lambda i: (i, 0, 0)),
                     pl.BlockSpec((1, 1), lambda i: (0, 0))]
        args += [dur.reshape(B, 1, T), ml]
        body = _pitch_body
    in_specs += [pl.BlockSpec((M, 1), lambda i: (i, 0)), qspec, qspec,
                 pl.BlockSpec((nb, Dd), lambda i: (0, 0))] + _wspecs(Dd)
    args += [tgt.reshape(R, 1), q1, q2, tab] + list(_wargs(p))
    pred, xout = pl.pallas_call(
        body,
        grid=(B,),
        in_specs=in_specs,
        out_specs=[pl.BlockSpec((1, 1, M), lambda i: (i, 0, 0)), xspec],
        out_shape=[jax.ShapeDtypeStruct((B, 1, M), jnp.float32),
                   jax.ShapeDtypeStruct((R, Dd), jnp.float32)],
    )(*args)
    return pred.reshape(B, M), xout


def kernel(x, duration_target, max_len, pitch_target, energy_target, params,
           pitch_quant, energy_quant):
    B, T, Dd = x.shape
    M = pitch_target.shape[1]
    x2d = x.reshape(B * T, Dd)
    log_dur = _dur_call(x2d, B, params['dur'])
    idx = _route_call(duration_target, M)
    xe0 = _lr_expand_sc(x2d, idx.reshape(-1, 128))
    pitch_pred, xe1 = _var_call(xe0, B, duration_target, max_len, pitch_target,
                                pitch_quant, params['pitch_tab'], params['pitch'])
    en_pred, xe2 = _var_call(xe1, B, None, None, energy_target,
                             energy_quant, params['energy_tab'], params['energy'])
    return (xe2.reshape(B, M, Dd), pitch_pred, en_pred, log_dur,
            duration_target, duration_target)


# speedup vs baseline: 1.0992x; 1.0992x over previous
import functools

import jax
import jax.numpy as jnp
from jax import lax
from jax.experimental import pallas as pl
from jax.experimental.pallas import tpu as pltpu
from jax.experimental.pallas import tpu_sc as plsc


def _lr_expand_sc(x2d, idx2d):
    NCH, CH = idx2d.shape
    Dd = x2d.shape[1]
    R = NCH * CH
    info = plsc.get_sparse_core_info()
    NC, NS = info.num_cores, info.num_subcores
    NW = NC * NS
    CPW = NCH // NW
    mesh = plsc.VectorSubcoreMesh(core_axis_name="c", subcore_axis_name="s")

    S = x2d.shape[0]
    SPT = S // NS
    G = 16
    scratch = [pltpu.VMEM_SHARED((S, Dd), jnp.float32),
               pltpu.VMEM((CPW, CH), jnp.int32)]
    scratch += [pltpu.VMEM((CH, Dd), jnp.float32) for _ in range(CPW)]
    scratch += [pltpu.SemaphoreType.DMA for _ in range(2 * CPW)]

    @functools.partial(
        pl.kernel, mesh=mesh,
        out_type=jax.ShapeDtypeStruct((R, Dd), jnp.float32),
        scratch_types=scratch,
        compiler_params=pltpu.CompilerParams(needs_layout_passes=False),
    )
    def k(x_hbm, idx_hbm, out_hbm, x_sh, idx_v, *bufs_and_sems):
        rows = bufs_and_sems[:CPW]
        sg = bufs_and_sems[CPW:2 * CPW]
        so = bufs_and_sems[2 * CPW:]
        sid = lax.axis_index("s")
        wid = sid * NC + lax.axis_index("c")
        c0 = wid * CPW
        pltpu.sync_copy(x_hbm.at[pl.ds(sid * SPT, SPT)],
                        x_sh.at[pl.ds(sid * SPT, SPT)])
        pltpu.sync_copy(idx_hbm.at[pl.ds(c0, CPW)], idx_v)
        plsc.subcore_barrier()
        outs = []
        for ci in range(CPW):
            pend = []
            for g in range(CH // G):
                v = idx_v[ci, pl.ds(g * G, G)]
                grp = [pltpu.async_copy(
                           x_sh.at[pl.ds(v[j], 1)],
                           rows[ci].at[pl.ds(g * G + j, 1)], sg[ci])
                       for j in range(G)]
                for p in pend:
                    p.wait()
                pend = grp
            for p in pend:
                p.wait()
            outs.append(pltpu.async_copy(
                rows[ci], out_hbm.at[pl.ds((c0 + ci) * CH, CH)], so[ci]))
        for o in outs:
            o.wait()

    return k(x2d, idx2d)


def _route_body(dur_ref, out_ref):
    t = dur_ref.shape[1]
    m = out_ref.shape[2]
    b = pl.program_id(0)
    dcol = dur_ref[0].astype(jnp.float32)
    ii = lax.broadcasted_iota(jnp.int32, (t, t), 0)
    jj = lax.broadcasted_iota(jnp.int32, (t, t), 1)
    tril = (jj <= ii).astype(jnp.float32)
    cumt = jnp.dot(tril, dcol, preferred_element_type=jnp.float32)
    frames = lax.broadcasted_iota(jnp.int32, (1, m), 1).astype(jnp.float32)
    cmpt = (cumt <= frames).astype(jnp.float32)
    ones = jnp.ones((1, t), jnp.float32)
    tok = jnp.dot(ones, cmpt, preferred_element_type=jnp.float32).astype(jnp.int32)
    out_ref[0] = jnp.minimum(tok, t - 1) + b * t


def _route_call(dur, M):
    B, T = dur.shape
    out = pl.pallas_call(
        _route_body,
        grid=(B,),
        in_specs=[pl.BlockSpec((1, T, 1), lambda i: (i, 0, 0))],
        out_specs=pl.BlockSpec((1, 1, M), lambda i: (i, 0, 0)),
        out_shape=jax.ShapeDtypeStruct((B, 1, M), jnp.int32),
    )(dur.reshape(B, T, 1))
    return out.reshape(B, M)


def _ln(h, s, b):
    mu = jnp.mean(h, axis=1, keepdims=True)
    d = h - mu
    var = jnp.mean(d * d, axis=1, keepdims=True)
    return d * lax.rsqrt(var + 1e-5) * s + b


def _conv3(h, w_ref, b_ref):
    a = jnp.dot(h, w_ref[0], preferred_element_type=jnp.float32)
    c = jnp.dot(h, w_ref[1], preferred_element_type=jnp.float32)
    e = jnp.dot(h, w_ref[2], preferred_element_type=jnp.float32)
    n = h.shape[0]
    z = jnp.zeros((1, a.shape[1]), jnp.float32)
    return (jnp.concatenate([z, a[: n - 1]], axis=0) + c
            + jnp.concatenate([e[1:], z], axis=0) + b_ref[...])


def _mlp(h, w1, b1, s1, g1, w2, b2, s2, g2, wl, bl):
    h = _ln(jnp.maximum(_conv3(h, w1, b1), 0.0), s1[...], g1[...])
    h = _ln(jnp.maximum(_conv3(h, w2, b2), 0.0), s2[...], g2[...])
    return jnp.dot(h, wl[...], preferred_element_type=jnp.float32) + bl[...]


def _wargs(p):
    d = p['b1'].shape[0]
    return (p['w1'], p['b1'].reshape(1, d), p['ln1_s'].reshape(1, d),
            p['ln1_b'].reshape(1, d), p['w2'], p['b2'].reshape(1, d),
            p['ln2_s'].reshape(1, d), p['ln2_b'].reshape(1, d),
            p['wl'], p['bl'].reshape(1, 1))


def _wspecs(d):
    def wspec(shape):
        return pl.BlockSpec(shape, lambda i: tuple(0 for _ in shape))
    return [wspec((3, d, d)), wspec((1, d)), wspec((1, d)), wspec((1, d)),
            wspec((3, d, d)), wspec((1, d)), wspec((1, d)), wspec((1, d)),
            wspec((d, 1)), wspec((1, 1))]


def _dur_body(x_ref, w1, b1, s1, g1, w2, b2, s2, g2, wl, bl, out_ref):
    out_ref[...] = _mlp(x_ref[...], w1, b1, s1, g1, w2, b2, s2, g2, wl, bl)


def _dur_call(x2d, B, p):
    R, Dd = x2d.shape
    T = R // B
    out = pl.pallas_call(
        _dur_body,
        grid=(B,),
        in_specs=[pl.BlockSpec((T, Dd), lambda i: (i, 0))] + _wspecs(Dd),
        out_specs=pl.BlockSpec((T, 1), lambda i: (i, 0)),
        out_shape=jax.ShapeDtypeStruct((R, 1), jnp.float32),
    )(x2d, *_wargs(p))
    return out.reshape(B, T)


def _emb_add(tgt_ref, q1_ref, q2_ref, tab_ref):
    v = tgt_ref[...]
    oh = ((q1_ref[...] < v) & (v <= q2_ref[...])).astype(jnp.float32)
    return jnp.dot(oh, tab_ref[...], preferred_element_type=jnp.float32)


def _pitch_body(xe_ref, dur_ref, ml_ref, tgt_ref, q1_ref, q2_ref, tab_ref,
                w1, b1, s1, g1, w2, b2, s2, g2, wl, bl, pred_ref, xout_ref):
    m = xe_ref.shape[0]
    bound = jnp.minimum(jnp.sum(dur_ref[...]), ml_ref[0, 0])
    io = lax.broadcasted_iota(jnp.int32, (m, 1), 0)
    h = xe_ref[...] * (io < bound).astype(jnp.float32)
    pred_ref[...] = _mlp(h, w1, b1, s1, g1, w2, b2, s2, g2, wl, bl)
    xout_ref[...] = h + _emb_add(tgt_ref, q1_ref, q2_ref, tab_ref)


def _energy_body(xe_ref, tgt_ref, q1_ref, q2_ref, tab_ref,
                 w1, b1, s1, g1, w2, b2, s2, g2, wl, bl, pred_ref, xout_ref):
    h = xe_ref[...]
    pred_ref[...] = _mlp(h, w1, b1, s1, g1, w2, b2, s2, g2, wl, bl)
    xout_ref[...] = h + _emb_add(tgt_ref, q1_ref, q2_ref, tab_ref)


def _quant_bounds(quant):
    q1 = jnp.concatenate([jnp.full((1,), -jnp.inf, jnp.float32), quant])
    q2 = jnp.concatenate([quant, jnp.full((1,), jnp.inf, jnp.float32)])
    return q1.reshape(1, -1), q2.reshape(1, -1)


def _var_call(xe2d, B, dur, max_len, tgt, quant, tab, p):
    R, Dd = xe2d.shape
    M = R // B
    nb = tab.shape[0]
    q1, q2 = _quant_bounds(quant)
    xspec = pl.BlockSpec((M, Dd), lambda i: (i, 0))
    qspec = pl.BlockSpec((1, nb), lambda i: (0, 0))
    in_specs = [xspec]
    args = [xe2d]
    body = _energy_body
    if dur is not None:
        T = dur.shape[1]
        ml = jnp.asarray(max_len, jnp.int32).reshape(1, 1)
        in_specs += [pl.BlockSpec((1, 1, T), lambda i: (i, 0, 0)),
                     pl.BlockSpec((1, 1), lambda i: (0, 0))]
        args += [dur.reshape(B, 1, T), ml]
        body = _pitch_body
    in_specs += [pl.BlockSpec((M, 1), lambda i: (i, 0)), qspec, qspec,
                 pl.BlockSpec((nb, Dd), lambda i: (0, 0))] + _wspecs(Dd)
    args += [tgt.reshape(R, 1), q1, q2, tab] + list(_wargs(p))
    pred, xout = pl.pallas_call(
        body,
        grid=(B,),
        in_specs=in_specs,
        out_specs=[pl.BlockSpec((M, 1), lambda i: (i, 0)), xspec],
        out_shape=[jax.ShapeDtypeStruct((R, 1), jnp.float32),
                   jax.ShapeDtypeStruct((R, Dd), jnp.float32)],
    )(*args)
    return pred.reshape(B, M), xout


def kernel(x, duration_target, max_len, pitch_target, energy_target, params,
           pitch_quant, energy_quant):
    B, T, Dd = x.shape
    M = pitch_target.shape[1]
    x2d = x.reshape(B * T, Dd)
    log_dur = _dur_call(x2d, B, params['dur'])
    idx = _route_call(duration_target, M)
    xe0 = _lr_expand_sc(x2d, idx.reshape(-1, 128))
    pitch_pred, xe1 = _var_call(xe0, B, duration_target, max_len, pitch_target,
                                pitch_quant, params['pitch_tab'], params['pitch'])
    en_pred, xe2 = _var_call(xe1, B, None, None, energy_target,
                             energy_quant, params['energy_tab'], params['energy'])
    return (xe2.reshape(B, M, Dd), pitch_pred, en_pred, log_dur,
            duration_target, duration_target)


# speedup vs baseline: 1.1378x; 1.0351x over previous
import functools

import jax
import jax.numpy as jnp
from jax import lax
from jax.experimental import pallas as pl
from jax.experimental.pallas import tpu as pltpu
from jax.experimental.pallas import tpu_sc as plsc


def _lr_expand_sc(x2d, idx2d):
    NCH, CH = idx2d.shape
    Dd = x2d.shape[1]
    R = NCH * CH
    info = plsc.get_sparse_core_info()
    NC, NS = info.num_cores, info.num_subcores
    NW = NC * NS
    CPW = NCH // NW
    mesh = plsc.VectorSubcoreMesh(core_axis_name="c", subcore_axis_name="s")

    S = x2d.shape[0]
    SPT = S // NS
    G = 16
    scratch = [pltpu.VMEM_SHARED((S, Dd), jnp.float32),
               pltpu.VMEM((CPW, CH), jnp.int32)]
    scratch += [pltpu.VMEM((CH, Dd), jnp.float32) for _ in range(CPW)]
    scratch += [pltpu.SemaphoreType.DMA for _ in range(2 * CPW)]

    @functools.partial(
        pl.kernel, mesh=mesh,
        out_type=jax.ShapeDtypeStruct((R, Dd), jnp.float32),
        scratch_types=scratch,
        compiler_params=pltpu.CompilerParams(needs_layout_passes=False),
    )
    def k(x_hbm, idx_hbm, out_hbm, x_sh, idx_v, *bufs_and_sems):
        rows = bufs_and_sems[:CPW]
        sg = bufs_and_sems[CPW:2 * CPW]
        so = bufs_and_sems[2 * CPW:]
        sid = lax.axis_index("s")
        wid = sid * NC + lax.axis_index("c")
        c0 = wid * CPW
        pltpu.sync_copy(x_hbm.at[pl.ds(sid * SPT, SPT)],
                        x_sh.at[pl.ds(sid * SPT, SPT)])
        pltpu.sync_copy(idx_hbm.at[pl.ds(c0, CPW)], idx_v)
        plsc.subcore_barrier()
        outs = []
        for ci in range(CPW):
            pend = []
            for g in range(CH // G):
                v = idx_v[ci, pl.ds(g * G, G)]
                grp = [pltpu.async_copy(
                           x_sh.at[pl.ds(v[j], 1)],
                           rows[ci].at[pl.ds(g * G + j, 1)], sg[ci])
                       for j in range(G)]
                for p in pend:
                    p.wait()
                pend = grp
            for p in pend:
                p.wait()
            outs.append(pltpu.async_copy(
                rows[ci], out_hbm.at[pl.ds((c0 + ci) * CH, CH)], so[ci]))
        for o in outs:
            o.wait()

    return k(x2d, idx2d)


def _route_body(dur_ref, out_ref):
    t = dur_ref.shape[1]
    m = out_ref.shape[2]
    b = pl.program_id(0)
    dcol = dur_ref[0].astype(jnp.float32)
    ii = lax.broadcasted_iota(jnp.int32, (t, t), 0)
    jj = lax.broadcasted_iota(jnp.int32, (t, t), 1)
    tril = (jj <= ii).astype(jnp.float32)
    cumt = jnp.dot(tril, dcol, preferred_element_type=jnp.float32)
    frames = lax.broadcasted_iota(jnp.int32, (1, m), 1).astype(jnp.float32)
    cmpt = (cumt <= frames).astype(jnp.float32)
    ones = jnp.ones((1, t), jnp.float32)
    tok = jnp.dot(ones, cmpt, preferred_element_type=jnp.float32).astype(jnp.int32)
    out_ref[0] = jnp.minimum(tok, t - 1) + b * t


def _route_call(dur, M):
    B, T = dur.shape
    out = pl.pallas_call(
        _route_body,
        grid=(B,),
        in_specs=[pl.BlockSpec((1, T, 1), lambda i: (i, 0, 0))],
        out_specs=pl.BlockSpec((1, 1, M), lambda i: (i, 0, 0)),
        out_shape=jax.ShapeDtypeStruct((B, 1, M), jnp.int32),
    )(dur.reshape(B, T, 1))
    return out.reshape(B, M)


def _ln(h, s, b):
    mu = jnp.mean(h, axis=1, keepdims=True)
    d = h - mu
    var = jnp.mean(d * d, axis=1, keepdims=True)
    return d * lax.rsqrt(var + 1e-5) * s + b


def _conv3(h, w_ref, b_ref):
    a = jnp.dot(h, w_ref[0], preferred_element_type=jnp.float32)
    c = jnp.dot(h, w_ref[1], preferred_element_type=jnp.float32)
    e = jnp.dot(h, w_ref[2], preferred_element_type=jnp.float32)
    n = h.shape[0]
    z = jnp.zeros((1, a.shape[1]), jnp.float32)
    return (jnp.concatenate([z, a[: n - 1]], axis=0) + c
            + jnp.concatenate([e[1:], z], axis=0) + b_ref[...])


def _mlp(h, w1, b1, s1, g1, w2, b2, s2, g2, wl, bl):
    h = _ln(jnp.maximum(_conv3(h, w1, b1), 0.0), s1[...], g1[...])
    h = _ln(jnp.maximum(_conv3(h, w2, b2), 0.0), s2[...], g2[...])
    return jnp.dot(h, wl[...], preferred_element_type=jnp.float32) + bl[...]


def _wargs(p):
    d = p['b1'].shape[0]
    return (p['w1'], p['b1'].reshape(1, d), p['ln1_s'].reshape(1, d),
            p['ln1_b'].reshape(1, d), p['w2'], p['b2'].reshape(1, d),
            p['ln2_s'].reshape(1, d), p['ln2_b'].reshape(1, d),
            p['wl'], p['bl'].reshape(1, 1))


def _wspecs(d):
    def wspec(shape):
        return pl.BlockSpec(shape, lambda i: tuple(0 for _ in shape))
    return [wspec((3, d, d)), wspec((1, d)), wspec((1, d)), wspec((1, d)),
            wspec((3, d, d)), wspec((1, d)), wspec((1, d)), wspec((1, d)),
            wspec((d, 1)), wspec((1, 1))]


def _dur_body(x_ref, w1, b1, s1, g1, w2, b2, s2, g2, wl, bl, out_ref):
    out_ref[...] = _mlp(x_ref[...], w1, b1, s1, g1, w2, b2, s2, g2, wl, bl)


def _dur_call(x2d, B, p):
    R, Dd = x2d.shape
    T = R // B
    out = pl.pallas_call(
        _dur_body,
        grid=(B,),
        in_specs=[pl.BlockSpec((T, Dd), lambda i: (i, 0))] + _wspecs(Dd),
        out_specs=pl.BlockSpec((T, 1), lambda i: (i, 0)),
        out_shape=jax.ShapeDtypeStruct((R, 1), jnp.float32),
    )(x2d, *_wargs(p))
    return out.reshape(B, T)


def _emb_add(tgt_ref, q1_ref, q2_ref, tab_ref):
    v = tgt_ref[...]
    oh = ((q1_ref[...] < v) & (v <= q2_ref[...])).astype(jnp.float32)
    return jnp.dot(oh, tab_ref[...], preferred_element_type=jnp.float32)


def _var_body(xe_ref, dur_ref, ml_ref, ptgt_ref, pq1_ref, pq2_ref, ptab_ref,
              etgt_ref, eq1_ref, eq2_ref, etab_ref, *rest):
    pw = rest[:10]
    ew = rest[10:20]
    ppred_ref, epred_ref, xout_ref = rest[20:]
    m = xe_ref.shape[0]
    bound = jnp.minimum(jnp.sum(dur_ref[...]), ml_ref[0, 0])
    io = lax.broadcasted_iota(jnp.int32, (m, 1), 0)
    h = xe_ref[...] * (io < bound).astype(jnp.float32)
    ppred_ref[...] = _mlp(h, *pw)
    x1 = h + _emb_add(ptgt_ref, pq1_ref, pq2_ref, ptab_ref)
    epred_ref[...] = _mlp(x1, *ew)
    xout_ref[...] = x1 + _emb_add(etgt_ref, eq1_ref, eq2_ref, etab_ref)


def _quant_bounds(quant):
    q1 = jnp.concatenate([jnp.full((1,), -jnp.inf, jnp.float32), quant])
    q2 = jnp.concatenate([quant, jnp.full((1,), jnp.inf, jnp.float32)])
    return q1.reshape(1, -1), q2.reshape(1, -1)


def _var_call(xe2d, B, dur, max_len, ptgt, pquant, ptab, pp,
              etgt, equant, etab, ep):
    R, Dd = xe2d.shape
    M = R // B
    T = dur.shape[1]
    nb = ptab.shape[0]
    pq1, pq2 = _quant_bounds(pquant)
    eq1, eq2 = _quant_bounds(equant)
    ml = jnp.asarray(max_len, jnp.int32).reshape(1, 1)
    xspec = pl.BlockSpec((M, Dd), lambda i: (i, 0))
    qspec = pl.BlockSpec((1, nb), lambda i: (0, 0))
    tspec = pl.BlockSpec((M, 1), lambda i: (i, 0))
    tabspec = pl.BlockSpec((nb, Dd), lambda i: (0, 0))
    in_specs = [xspec,
                pl.BlockSpec((1, 1, T), lambda i: (i, 0, 0)),
                pl.BlockSpec((1, 1), lambda i: (0, 0)),
                tspec, qspec, qspec, tabspec,
                tspec, qspec, qspec, tabspec] + _wspecs(Dd) + _wspecs(Dd)
    args = [xe2d, dur.reshape(B, 1, T), ml,
            ptgt.reshape(R, 1), pq1, pq2, ptab,
            etgt.reshape(R, 1), eq1, eq2, etab] + list(_wargs(pp)) + list(_wargs(ep))
    ppred, epred, xout = pl.pallas_call(
        _var_body,
        grid=(B,),
        in_specs=in_specs,
        out_specs=[tspec, tspec, xspec],
        out_shape=[jax.ShapeDtypeStruct((R, 1), jnp.float32),
                   jax.ShapeDtypeStruct((R, 1), jnp.float32),
                   jax.ShapeDtypeStruct((R, Dd), jnp.float32)],
    )(*args)
    return ppred.reshape(B, M), epred.reshape(B, M), xout


def kernel(x, duration_target, max_len, pitch_target, energy_target, params,
           pitch_quant, energy_quant):
    B, T, Dd = x.shape
    M = pitch_target.shape[1]
    x2d = x.reshape(B * T, Dd)
    log_dur = _dur_call(x2d, B, params['dur'])
    idx = _route_call(duration_target, M)
    xe0 = _lr_expand_sc(x2d, idx.reshape(-1, 128))
    pitch_pred, en_pred, xe2 = _var_call(
        xe0, B, duration_target, max_len,
        pitch_target, pitch_quant, params['pitch_tab'], params['pitch'],
        energy_target, energy_quant, params['energy_tab'], params['energy'])
    return (xe2.reshape(B, M, Dd), pitch_pred, en_pred, log_dur,
            duration_target, duration_target)
